# baseline (speedup 1.0000x reference)
import jax
from jax.experimental import pallas as pl
from jax.experimental.pallas import tpu as pltpu

_N_STRIPES = 8


def _copy_kernel(x_ref, o_ref, *sems):
    n = x_ref.shape[0]
    stripe = n // _N_STRIPES
    copies = []
    for i in range(_N_STRIPES):
        sl = pl.ds(i * stripe, stripe)
        c = pltpu.make_async_copy(x_ref.at[sl], o_ref.at[sl], sems[i])
        c.start()
        copies.append(c)
    for c in copies:
        c.wait()


def kernel(x, edge_index):
    del edge_index
    return pl.pallas_call(
        _copy_kernel,
        out_shape=jax.ShapeDtypeStruct(x.shape, x.dtype),
        in_specs=[pl.BlockSpec(memory_space=pl.ANY)],
        out_specs=pl.BlockSpec(memory_space=pl.ANY),
        scratch_shapes=[pltpu.SemaphoreType.DMA] * _N_STRIPES,
    )(x)


# speedup vs baseline: 23.8842x; 23.8842x over previous
import jax
from jax.experimental import pallas as pl
from jax.experimental.pallas import tpu as pltpu

_BLOCK_ROWS = 2000


def _copy_kernel(x_ref, o_ref):
    o_ref[...] = x_ref[...]


def kernel(x, edge_index):
    del edge_index
    n, d = x.shape
    grid = (pl.cdiv(n, _BLOCK_ROWS),)
    return pl.pallas_call(
        _copy_kernel,
        out_shape=jax.ShapeDtypeStruct(x.shape, x.dtype),
        grid=grid,
        in_specs=[pl.BlockSpec((_BLOCK_ROWS, d), lambda i: (i, 0))],
        out_specs=pl.BlockSpec((_BLOCK_ROWS, d), lambda i: (i, 0)),
        compiler_params=pltpu.CompilerParams(
            dimension_semantics=("arbitrary",),
        ),
    )(x)


# speedup vs baseline: 37.3040x; 1.5619x over previous
import jax
from jax.experimental import pallas as pl
from jax.experimental.pallas import tpu as pltpu

_BLOCK_ROWS = 5000


def _copy_kernel(x_ref, o_ref):
    o_ref[...] = x_ref[...]


def kernel(x, edge_index):
    del edge_index
    n, d = x.shape
    grid = (pl.cdiv(n, _BLOCK_ROWS),)
    return pl.pallas_call(
        _copy_kernel,
        out_shape=jax.ShapeDtypeStruct(x.shape, x.dtype),
        grid=grid,
        in_specs=[pl.BlockSpec((_BLOCK_ROWS, d), lambda i: (i, 0))],
        out_specs=pl.BlockSpec((_BLOCK_ROWS, d), lambda i: (i, 0)),
        compiler_params=pltpu.CompilerParams(
            dimension_semantics=("arbitrary",),
        ),
    )(x)
